# baseline (speedup 1.0000x reference)
import functools

import jax
import jax.numpy as jnp
from jax import lax
from jax.experimental import pallas as pl
from jax.experimental.pallas import tpu as pltpu
from jax.experimental.pallas import tpu_sc as plsc

_NUM_MASHUP = 100000
_NUM_API = 1000
_BATCH = 4096

_L = 16
_NW = 32
_B_PER_W = _BATCH // _NW
_G = 4
_NPAIR = _B_PER_W // (2 * _G)
_FULL = _NUM_API // _L
_TAIL = _NUM_API - _L
_SPLIT = 496
_LASTB = (_NUM_MASHUP // 128) * 128
_ROW = 1016


def _sigmoid16(x):
    return 1.0 / (1.0 + jnp.exp(-x))


def _s1_body(su_hbm, tt_hbm, m3_hbm, idx_v, buf_a, buf_b,
             oslab_a, oslab_b, gsem_a, gsem_b, osem_a, osem_b):
    wid = lax.axis_index("s") * 2 + lax.axis_index("c")
    base = pl.multiple_of(wid * _B_PER_W, 8)
    lane = lax.iota(jnp.int32, _L)

    def row(p, r_cur, oslab, osem):
        @pl.when(lax.bitwise_and(p, 15) == 0)
        def _():
            off = pl.multiple_of(
                base + lax.shift_left(lax.shift_right_logical(p, 4), 4), 8)
            pltpu.sync_copy(su_hbm.at[pl.ds(off, _L)], idx_v)

        x = idx_v[pl.ds(0, _L)]
        u_p = jnp.sum(jnp.where(lane == lax.bitwise_and(p, 15), x, 0))
        r_new = lax.shift_right_logical(u_p, 7)
        l_in = lax.bitwise_and(u_p, 127)
        fetch = jnp.logical_and(r_new != r_cur, r_new <= 780)

        @pl.when(fetch)
        def _():
            col = pl.multiple_of(lax.shift_left(r_new, 7), 128)
            pltpu.async_copy(tt_hbm.at[pl.ds(0, _SPLIT), pl.ds(col, 128)],
                             buf_a, gsem_a)
            pltpu.async_copy(
                tt_hbm.at[pl.ds(_SPLIT, _NUM_API - _SPLIT), pl.ds(col, 128)],
                buf_b, gsem_b)

        @pl.when(p >= 2)
        def _():
            pltpu.make_async_copy(
                oslab, m3_hbm.at[pl.ds(0, _ROW)], osem).wait()

        @pl.when(fetch)
        def _():
            pltpu.make_async_copy(
                tt_hbm.at[pl.ds(0, _SPLIT), pl.ds(0, 128)], buf_a,
                gsem_a).wait()

        @pl.when(r_new <= 780)
        def _():
            ls = jnp.full((_L,), l_in, jnp.int32)

            def chunk_a(c):
                v = plsc.load_gather(buf_a, [c * _L + lane, ls])
                oslab[pl.ds(lax.shift_left(c, 4), _L)] = v

            plsc.parallel_loop(0, _SPLIT // _L, unroll=4)(chunk_a)

        @pl.when(fetch)
        def _():
            pltpu.make_async_copy(
                tt_hbm.at[pl.ds(_SPLIT, _NUM_API - _SPLIT), pl.ds(0, 128)],
                buf_b, gsem_b).wait()

        @pl.when(r_new <= 780)
        def _():
            ls = jnp.full((_L,), l_in, jnp.int32)

            def chunk_b(c):
                v = plsc.load_gather(buf_b, [c * _L - _SPLIT + lane, ls])
                oslab[pl.ds(lax.shift_left(c, 4), _L)] = v

            plsc.parallel_loop(_SPLIT // _L, _FULL, unroll=4)(chunk_b)
            v = plsc.load_gather(buf_b, [_TAIL - _SPLIT + lane, ls])
            oslab[pl.ds(_TAIL, _L)] = v

        oslab[pl.ds(_NUM_API, _L)] = plsc.bitcast(
            jnp.full((_L,), u_p, jnp.int32), jnp.float32)

        pr = pl.multiple_of((base + p) * _ROW, 8)
        pltpu.async_copy(oslab, m3_hbm.at[pl.ds(pr, _ROW)], osem)
        return r_new

    def rowpair(i, r_cur):
        r_cur = row(2 * i, r_cur, oslab_a, osem_a)
        r_cur = row(2 * i + 1, r_cur, oslab_b, osem_b)
        return r_cur

    lax.fori_loop(0, _B_PER_W // 2, rowpair, jnp.int32(-1))
    pltpu.make_async_copy(oslab_a, m3_hbm.at[pl.ds(0, _ROW)], osem_a).wait()
    pltpu.make_async_copy(oslab_b, m3_hbm.at[pl.ds(0, _ROW)], osem_b).wait()


def _s2_body(rank_hbm, m3_hbm, tt_hbm, out_hbm,
             idx_v, buf_a, buf_b, buf_l, oslab, gsem_a, gsem_b, osem):
    wid = lax.axis_index("s") * 2 + lax.axis_index("c")
    base = pl.multiple_of(wid * _B_PER_W, 8)
    pltpu.sync_copy(rank_hbm.at[pl.ds(base, _B_PER_W)],
                    idx_v.at[pl.ds(0, _B_PER_W)])
    lane = lax.iota(jnp.int32, _L)

    def fire4(x, lane_off, buf, sem):
        for jj in range(_G):
            r_j = jnp.sum(jnp.where(lane == lane_off + jj, x, 0))
            off = pl.multiple_of(r_j * _ROW, 8)
            pltpu.async_copy(m3_hbm.at[pl.ds(off, _ROW)],
                             buf.at[pl.ds(jj * _ROW, _ROW)], sem)

    def drain4(buf, sem):
        for jj in range(_G):
            pltpu.make_async_copy(
                m3_hbm.at[pl.ds(0, _ROW)],
                buf.at[pl.ds(jj * _ROW, _ROW)], sem).wait()

    def compute4(buf, orow0):
        for jj in range(_G):
            jbase = jnp.full((_L,), jj * _ROW, jnp.int32)

            def chunk(c):
                v = plsc.load_gather(buf, [jbase + c * _L + lane])
                oslab[orow0 + jj, pl.ds(c * _L, _L)] = _sigmoid16(v)

            plsc.parallel_loop(0, _FULL, unroll=4)(chunk)
            v = plsc.load_gather(buf, [jbase + _TAIL + lane])
            oslab[orow0 + jj, pl.ds(_TAIL, _L)] = _sigmoid16(v)

            vs = plsc.load_gather(buf, [jbase + _NUM_API])
            u_j = jnp.sum(jnp.where(
                lane == 0, plsc.bitcast(vs, jnp.int32), 0))

            @pl.when(u_j >= _LASTB)
            def _():
                lb = jnp.full((_L,), u_j - _LASTB, jnp.int32)
                l8 = lax.bitwise_and(lane, 7)

                def piece(kp2, carry):
                    o16 = pl.multiple_of(kp2 * _L, 8)
                    pltpu.sync_copy(
                        tt_hbm.at[pl.ds(o16, 8), pl.ds(_LASTB, 32)], buf_l)
                    v_lo = plsc.load_gather(buf_l, [l8, lb])
                    pltpu.sync_copy(
                        tt_hbm.at[pl.ds(o16 + 8, 8), pl.ds(_LASTB, 32)],
                        buf_l)
                    v_hi = plsc.load_gather(buf_l, [l8, lb])
                    vv = jnp.where(lane < 8, v_lo, v_hi)
                    oslab[orow0 + jj, pl.ds(o16, _L)] = _sigmoid16(vv)
                    return carry

                lax.fori_loop(0, 62, piece, 0)
                pltpu.sync_copy(
                    tt_hbm.at[pl.ds(984, 8), pl.ds(_LASTB, 32)], buf_l)
                v_lo = plsc.load_gather(buf_l, [l8, lb])
                pltpu.sync_copy(
                    tt_hbm.at[pl.ds(992, 8), pl.ds(_LASTB, 32)], buf_l)
                v_hi = plsc.load_gather(buf_l, [l8, lb])
                vv = jnp.where(lane < 8, v_lo, v_hi)
                oslab[orow0 + jj, pl.ds(_TAIL, _L)] = _sigmoid16(vv)

    x0 = idx_v[pl.ds(0, _L)]
    fire4(x0, 0, buf_a, gsem_a)

    def pair(i, carry):
        x = idx_v[pl.ds(pl.multiple_of(i * 8, 8), _L)]
        fire4(x, _G, buf_b, gsem_b)

        @pl.when(i > 0)
        def _():
            pltpu.make_async_copy(
                oslab, out_hbm.at[pl.ds(0, 2 * _G)], osem).wait()

        drain4(buf_a, gsem_a)
        compute4(buf_a, 0)

        xn = idx_v[pl.ds(pl.multiple_of(i * 8 + 8, 8), _L)]

        @pl.when(i < _NPAIR - 1)
        def _():
            fire4(xn, 0, buf_a, gsem_a)

        drain4(buf_b, gsem_b)
        compute4(buf_b, _G)

        off = pl.multiple_of(base + i * 8, 8)
        pltpu.async_copy(oslab, out_hbm.at[pl.ds(off, 2 * _G)], osem)
        return carry

    lax.fori_loop(0, _NPAIR, pair, 0)
    pltpu.make_async_copy(oslab, out_hbm.at[pl.ds(0, 2 * _G)], osem).wait()


@jax.jit
def _pop_sc(u, m2a_mat):
    table_t = m2a_mat.T
    iota = lax.iota(jnp.int32, _BATCH)
    sorted_u, perm = lax.sort_key_val(u, iota)
    rank = jnp.zeros((_BATCH,), jnp.int32).at[perm].set(iota)

    mesh = plsc.VectorSubcoreMesh(core_axis_name="c", subcore_axis_name="s")
    s1 = functools.partial(
        pl.kernel,
        mesh=mesh,
        compiler_params=pltpu.CompilerParams(needs_layout_passes=False),
        out_type=jax.ShapeDtypeStruct((_BATCH * _ROW,), jnp.float32),
        scratch_types=[
            pltpu.VMEM((_L,), jnp.int32),
            pltpu.VMEM((_SPLIT, 128), jnp.float32),
            pltpu.VMEM((_NUM_API - _SPLIT, 128), jnp.float32),
            pltpu.VMEM((_ROW,), jnp.float32),
            pltpu.VMEM((_ROW,), jnp.float32),
            pltpu.SemaphoreType.DMA,
            pltpu.SemaphoreType.DMA,
            pltpu.SemaphoreType.DMA,
            pltpu.SemaphoreType.DMA,
        ],
    )(_s1_body)
    m3 = s1(sorted_u, table_t)

    s2 = functools.partial(
        pl.kernel,
        mesh=mesh,
        compiler_params=pltpu.CompilerParams(needs_layout_passes=False),
        out_type=jax.ShapeDtypeStruct((_BATCH, _NUM_API), jnp.float32),
        scratch_types=[
            pltpu.VMEM((_B_PER_W + _L,), jnp.int32),
            pltpu.VMEM((_G * _ROW,), jnp.float32),
            pltpu.VMEM((_G * _ROW,), jnp.float32),
            pltpu.VMEM((8, 32), jnp.float32),
            pltpu.VMEM((2 * _G, _NUM_API), jnp.float32),
            pltpu.SemaphoreType.DMA,
            pltpu.SemaphoreType.DMA,
            pltpu.SemaphoreType.DMA,
        ],
    )(_s2_body)
    return s2(rank, m3, table_t)


def kernel(u, m2a_mat):
    return _pop_sc(u, m2a_mat)


# speedup vs baseline: 1.1317x; 1.1317x over previous
import functools

import jax
import jax.numpy as jnp
from jax import lax
from jax.experimental import pallas as pl
from jax.experimental.pallas import tpu as pltpu
from jax.experimental.pallas import tpu_sc as plsc

_NUM_MASHUP = 100000
_NUM_API = 1000
_BATCH = 4096

_L = 16
_NW = 32
_B_PER_W = _BATCH // _NW
_G = 4
_NPAIR = _B_PER_W // (2 * _G)
_FULL = _NUM_API // _L
_TAIL = _NUM_API - _L
_SPLIT = 496
_LASTB = (_NUM_MASHUP // 128) * 128
_ROW = 1016


def _sigmoid16(x):
    return 1.0 / (1.0 + jnp.exp(-x))


def _s1_body(su_hbm, tt_hbm, m3_hbm, idx_v, buf_a, buf_b,
             oslab_a, oslab_b, gsem_a, gsem_b, osem_a, osem_b):
    wid = lax.axis_index("s") * 2 + lax.axis_index("c")
    base = pl.multiple_of(wid * _B_PER_W, 8)
    lane = lax.iota(jnp.int32, _L)

    def row(p, r_cur, oslab, osem):
        @pl.when(lax.bitwise_and(p, 15) == 0)
        def _():
            off = pl.multiple_of(
                base + lax.shift_left(lax.shift_right_logical(p, 4), 4), 8)
            pltpu.sync_copy(su_hbm.at[pl.ds(off, _L)], idx_v)

        x = idx_v[pl.ds(0, _L)]
        u_p = jnp.sum(jnp.where(lane == lax.bitwise_and(p, 15), x, 0))
        r_new = lax.shift_right_logical(u_p, 7)
        l_in = lax.bitwise_and(u_p, 127)
        fetch = jnp.logical_and(r_new != r_cur, r_new <= 780)

        @pl.when(fetch)
        def _():
            col = pl.multiple_of(lax.shift_left(r_new, 7), 128)
            pltpu.async_copy(tt_hbm.at[pl.ds(0, _SPLIT), pl.ds(col, 128)],
                             buf_a, gsem_a)
            pltpu.async_copy(
                tt_hbm.at[pl.ds(_SPLIT, _NUM_API - _SPLIT), pl.ds(col, 128)],
                buf_b, gsem_b)

        @pl.when(p >= 2)
        def _():
            pltpu.make_async_copy(
                oslab, m3_hbm.at[pl.ds(0, _ROW)], osem).wait()

        @pl.when(fetch)
        def _():
            pltpu.make_async_copy(
                tt_hbm.at[pl.ds(0, _SPLIT), pl.ds(0, 128)], buf_a,
                gsem_a).wait()

        @pl.when(r_new <= 780)
        def _():
            ls = jnp.full((_L,), l_in, jnp.int32)

            def chunk_a(c):
                v = plsc.load_gather(buf_a, [c * _L + lane, ls])
                oslab[pl.ds(lax.shift_left(c, 4), _L)] = v

            plsc.parallel_loop(0, _SPLIT // _L, unroll=4)(chunk_a)

        @pl.when(fetch)
        def _():
            pltpu.make_async_copy(
                tt_hbm.at[pl.ds(_SPLIT, _NUM_API - _SPLIT), pl.ds(0, 128)],
                buf_b, gsem_b).wait()

        @pl.when(r_new <= 780)
        def _():
            ls = jnp.full((_L,), l_in, jnp.int32)

            def chunk_b(c):
                v = plsc.load_gather(buf_b, [c * _L - _SPLIT + lane, ls])
                oslab[pl.ds(lax.shift_left(c, 4), _L)] = v

            plsc.parallel_loop(_SPLIT // _L, _FULL, unroll=4)(chunk_b)
            v = plsc.load_gather(buf_b, [_TAIL - _SPLIT + lane, ls])
            oslab[pl.ds(_TAIL, _L)] = v

        oslab[pl.ds(_NUM_API, _L)] = plsc.bitcast(
            jnp.full((_L,), u_p, jnp.int32), jnp.float32)

        pr = pl.multiple_of((base + p) * _ROW, 8)
        pltpu.async_copy(oslab, m3_hbm.at[pl.ds(pr, _ROW)], osem)
        return r_new

    def rowpair(i, r_cur):
        r_cur = row(2 * i, r_cur, oslab_a, osem_a)
        r_cur = row(2 * i + 1, r_cur, oslab_b, osem_b)
        return r_cur

    lax.fori_loop(0, _B_PER_W // 2, rowpair, jnp.int32(-1))
    pltpu.make_async_copy(oslab_a, m3_hbm.at[pl.ds(0, _ROW)], osem_a).wait()
    pltpu.make_async_copy(oslab_b, m3_hbm.at[pl.ds(0, _ROW)], osem_b).wait()


def _s2_body(rank_hbm, m3_hbm, tt_hbm, out_hbm,
             idx_v, buf_a, buf_b, buf_l, oslab, gsem_a, gsem_b, osem):
    wid = lax.axis_index("s") * 2 + lax.axis_index("c")
    base = pl.multiple_of(wid * _B_PER_W, 8)
    pltpu.sync_copy(rank_hbm.at[pl.ds(base, _B_PER_W)],
                    idx_v.at[pl.ds(0, _B_PER_W)])
    lane = lax.iota(jnp.int32, _L)

    def fire4(x, lane_off, buf, sem):
        for jj in range(_G):
            r_j = jnp.sum(jnp.where(lane == lane_off + jj, x, 0))
            off = pl.multiple_of(r_j * _ROW, 8)
            pltpu.async_copy(m3_hbm.at[pl.ds(off, _ROW)],
                             buf.at[pl.ds(jj * _ROW, _ROW)], sem)

    def drain4(buf, sem):
        for jj in range(_G):
            pltpu.make_async_copy(
                m3_hbm.at[pl.ds(0, _ROW)],
                buf.at[pl.ds(jj * _ROW, _ROW)], sem).wait()

    def compute4(buf, orow0):
        for jj in range(_G):
            jbase = jnp.full((_L,), jj * _ROW, jnp.int32)

            def chunk(c):
                v = plsc.load_gather(buf, [jbase + c * _L + lane])
                oslab[orow0 + jj, pl.ds(c * _L, _L)] = _sigmoid16(v)

            plsc.parallel_loop(0, _FULL, unroll=4)(chunk)
            v = plsc.load_gather(buf, [jbase + _TAIL + lane])
            oslab[orow0 + jj, pl.ds(_TAIL, _L)] = _sigmoid16(v)

            vs = plsc.load_gather(buf, [jbase + _NUM_API])
            u_j = jnp.sum(jnp.where(
                lane == 0, plsc.bitcast(vs, jnp.int32), 0))

            @pl.when(u_j >= _LASTB)
            def _():
                lb = jnp.full((_L,), u_j - _LASTB, jnp.int32)
                for off in (0, 256, 512, 744):
                    pltpu.sync_copy(
                        tt_hbm.at[pl.ds(off, 256), pl.ds(_LASTB, 32)], buf_l)

                    def piece(k, carry):
                        v = plsc.load_gather(buf_l, [k * _L + lane, lb])
                        oslab[orow0 + jj,
                              pl.ds(off + k * _L, _L)] = _sigmoid16(v)
                        return carry

                    lax.fori_loop(0, 16, piece, 0)

    x0 = idx_v[pl.ds(0, _L)]
    fire4(x0, 0, buf_a, gsem_a)

    def pair(i, carry):
        x = idx_v[pl.ds(pl.multiple_of(i * 8, 8), _L)]
        fire4(x, _G, buf_b, gsem_b)

        @pl.when(i > 0)
        def _():
            pltpu.make_async_copy(
                oslab, out_hbm.at[pl.ds(0, 2 * _G)], osem).wait()

        drain4(buf_a, gsem_a)
        compute4(buf_a, 0)

        xn = idx_v[pl.ds(pl.multiple_of(i * 8 + 8, 8), _L)]

        @pl.when(i < _NPAIR - 1)
        def _():
            fire4(xn, 0, buf_a, gsem_a)

        drain4(buf_b, gsem_b)
        compute4(buf_b, _G)

        off = pl.multiple_of(base + i * 8, 8)
        pltpu.async_copy(oslab, out_hbm.at[pl.ds(off, 2 * _G)], osem)
        return carry

    lax.fori_loop(0, _NPAIR, pair, 0)
    pltpu.make_async_copy(oslab, out_hbm.at[pl.ds(0, 2 * _G)], osem).wait()


@jax.jit
def _pop_sc(u, m2a_mat):
    table_t = m2a_mat.T
    iota = lax.iota(jnp.int32, _BATCH)
    sorted_u, perm = lax.sort_key_val(u, iota)
    rank = jnp.zeros((_BATCH,), jnp.int32).at[perm].set(iota)

    mesh = plsc.VectorSubcoreMesh(core_axis_name="c", subcore_axis_name="s")
    s1 = functools.partial(
        pl.kernel,
        mesh=mesh,
        compiler_params=pltpu.CompilerParams(needs_layout_passes=False),
        out_type=jax.ShapeDtypeStruct((_BATCH * _ROW,), jnp.float32),
        scratch_types=[
            pltpu.VMEM((_L,), jnp.int32),
            pltpu.VMEM((_SPLIT, 128), jnp.float32),
            pltpu.VMEM((_NUM_API - _SPLIT, 128), jnp.float32),
            pltpu.VMEM((_ROW,), jnp.float32),
            pltpu.VMEM((_ROW,), jnp.float32),
            pltpu.SemaphoreType.DMA,
            pltpu.SemaphoreType.DMA,
            pltpu.SemaphoreType.DMA,
            pltpu.SemaphoreType.DMA,
        ],
    )(_s1_body)
    m3 = s1(sorted_u, table_t)

    s2 = functools.partial(
        pl.kernel,
        mesh=mesh,
        compiler_params=pltpu.CompilerParams(needs_layout_passes=False),
        out_type=jax.ShapeDtypeStruct((_BATCH, _NUM_API), jnp.float32),
        scratch_types=[
            pltpu.VMEM((_B_PER_W + _L,), jnp.int32),
            pltpu.VMEM((_G * _ROW,), jnp.float32),
            pltpu.VMEM((_G * _ROW,), jnp.float32),
            pltpu.VMEM((256, 32), jnp.float32),
            pltpu.VMEM((2 * _G, _NUM_API), jnp.float32),
            pltpu.SemaphoreType.DMA,
            pltpu.SemaphoreType.DMA,
            pltpu.SemaphoreType.DMA,
        ],
    )(_s2_body)
    return s2(rank, m3, table_t)


def kernel(u, m2a_mat):
    return _pop_sc(u, m2a_mat)
